# baseline (speedup 1.0000x reference)
import jax
import jax.numpy as jnp
from jax.experimental import pallas as pl
from jax.experimental.pallas import tpu as pltpu

N = 10000
F_IN = 128
H = 64
C = 32
BI = 200
NI = N // BI


def _gcn_body(adj_ref, x_ref, W1_ref, b1_ref, W2_ref, b2_ref, out_ref,
              s1_ref, s2_ref):
    step = pl.program_id(0)

    @pl.when(step == 0)
    def _():
        s1_ref[...] = jnp.dot(x_ref[...], W1_ref[...],
                              preferred_element_type=jnp.float32)

    @pl.when(step < NI)
    def _():
        g = jnp.dot(adj_ref[...], s1_ref[...],
                    preferred_element_type=jnp.float32)
        h = jnp.maximum(g + b1_ref[...], 0.0)
        s2_ref[pl.ds(step * BI, BI), :] = jnp.dot(
            h, W2_ref[...], preferred_element_type=jnp.float32)

    @pl.when(step >= NI)
    def _():
        acc = jnp.dot(adj_ref[...], s2_ref[...],
                      preferred_element_type=jnp.float32)
        out_ref[...] = acc + b2_ref[...]


def kernel(x, adj, W1, b1, W2, b2):
    b1r = b1.reshape(1, H)
    b2r = b2.reshape(1, C)
    out = pl.pallas_call(
        _gcn_body,
        grid=(2 * NI,),
        in_specs=[
            pl.BlockSpec((BI, N), lambda i: (i % NI, 0)),
            pl.BlockSpec((N, F_IN), lambda i: (0, 0)),
            pl.BlockSpec((F_IN, H), lambda i: (0, 0)),
            pl.BlockSpec((1, H), lambda i: (0, 0)),
            pl.BlockSpec((H, C), lambda i: (0, 0)),
            pl.BlockSpec((1, C), lambda i: (0, 0)),
        ],
        out_specs=pl.BlockSpec((BI, C), lambda i: (i % NI, 0)),
        out_shape=jax.ShapeDtypeStruct((N, C), jnp.float32),
        scratch_shapes=[
            pltpu.VMEM((N, H), jnp.float32),
            pltpu.VMEM((N, C), jnp.float32),
        ],
    )(adj, x, W1, b1r, W2, b2r)
    return out


# speedup vs baseline: 1.0283x; 1.0283x over previous
import jax
import jax.numpy as jnp
from jax.experimental import pallas as pl
from jax.experimental.pallas import tpu as pltpu

N = 10000
F_IN = 128
H = 64
C = 32
BI = 400
NI = N // BI


def _gcn_body(adj_ref, x_ref, W1_ref, b1_ref, W2_ref, b2_ref, out_ref,
              s1_ref, s2_ref):
    step = pl.program_id(0)

    @pl.when(step == 0)
    def _():
        s1_ref[...] = jnp.dot(x_ref[...], W1_ref[...],
                              preferred_element_type=jnp.float32)

    @pl.when(step < NI)
    def _():
        g = jnp.dot(adj_ref[...], s1_ref[...],
                    preferred_element_type=jnp.float32)
        h = jnp.maximum(g + b1_ref[...], 0.0)
        s2_ref[pl.ds(step * BI, BI), :] = jnp.dot(
            h, W2_ref[...], preferred_element_type=jnp.float32)

    @pl.when(step >= NI)
    def _():
        acc = jnp.dot(adj_ref[...], s2_ref[...],
                      preferred_element_type=jnp.float32)
        out_ref[...] = acc + b2_ref[...]


def _stripe(i):
    return jnp.where(i < NI, i, 2 * NI - 1 - i)


def kernel(x, adj, W1, b1, W2, b2):
    b1r = b1.reshape(1, H)
    b2r = b2.reshape(1, C)
    out = pl.pallas_call(
        _gcn_body,
        grid=(2 * NI,),
        in_specs=[
            pl.BlockSpec((BI, N), lambda i: (_stripe(i), 0)),
            pl.BlockSpec((N, F_IN), lambda i: (0, 0)),
            pl.BlockSpec((F_IN, H), lambda i: (0, 0)),
            pl.BlockSpec((1, H), lambda i: (0, 0)),
            pl.BlockSpec((H, C), lambda i: (0, 0)),
            pl.BlockSpec((1, C), lambda i: (0, 0)),
        ],
        out_specs=pl.BlockSpec((BI, C), lambda i: (jnp.where(i < NI, 0, 2 * NI - 1 - i), 0)),
        out_shape=jax.ShapeDtypeStruct((N, C), jnp.float32),
        scratch_shapes=[
            pltpu.VMEM((N, H), jnp.float32),
            pltpu.VMEM((N, C), jnp.float32),
        ],
    )(adj, x, W1, b1r, W2, b2r)
    return out


# speedup vs baseline: 1.0344x; 1.0059x over previous
import jax
import jax.numpy as jnp
from jax import lax
from jax.experimental import pallas as pl
from jax.experimental.pallas import tpu as pltpu

N = 10000
F_IN = 128
H = 64
C = 32
BI = 200
NI = N // BI
NBUF = 5


def _fetch(adj_hbm, ring_ref, sems, stripe, slot):
    return pltpu.make_async_copy(
        adj_hbm.at[pl.ds(stripe * BI, BI), :], ring_ref.at[slot],
        sems.at[slot])


def _gcn_body(adj_hbm, x_ref, W1_ref, b1_ref, W2_ref, b2_ref, out_ref,
              ring_ref, s1_ref, s2_ref, sems):
    for b in range(NBUF):
        _fetch(adj_hbm, ring_ref, sems, b, b).start()

    s1_ref[...] = jnp.dot(x_ref[...], W1_ref[...],
                          preferred_element_type=jnp.float32)

    def ph1(i, carry):
        slot = lax.rem(i, NBUF)
        _fetch(adj_hbm, ring_ref, sems, i, slot).wait()
        g = jnp.dot(ring_ref[slot], s1_ref[...],
                    preferred_element_type=jnp.float32)
        h = jnp.maximum(g + b1_ref[...], 0.0)
        s2_ref[pl.ds(i * BI, BI), :] = jnp.dot(
            h, W2_ref[...], preferred_element_type=jnp.float32)

        @pl.when(i + NBUF < NI)
        def _():
            _fetch(adj_hbm, ring_ref, sems, i + NBUF, slot).start()
        return carry

    lax.fori_loop(0, NI, ph1, 0)

    def ph2(t, carry):
        j = NI - 1 - t
        slot = lax.rem(j, NBUF)

        @pl.when(t >= NBUF)
        def _():
            _fetch(adj_hbm, ring_ref, sems, j, slot).wait()

        acc = jnp.dot(ring_ref[slot], s2_ref[...],
                      preferred_element_type=jnp.float32)
        out_ref[pl.ds(j * BI, BI), :] = acc + b2_ref[...]

        @pl.when(j >= NBUF)
        def _():
            _fetch(adj_hbm, ring_ref, sems, j - NBUF, slot).start()
        return carry

    lax.fori_loop(0, NI, ph2, 0)


def kernel(x, adj, W1, b1, W2, b2):
    b1r = b1.reshape(1, H)
    b2r = b2.reshape(1, C)
    out = pl.pallas_call(
        _gcn_body,
        in_specs=[
            pl.BlockSpec(memory_space=pltpu.MemorySpace.HBM),
            pl.BlockSpec(memory_space=pltpu.VMEM),
            pl.BlockSpec(memory_space=pltpu.VMEM),
            pl.BlockSpec(memory_space=pltpu.VMEM),
            pl.BlockSpec(memory_space=pltpu.VMEM),
            pl.BlockSpec(memory_space=pltpu.VMEM),
        ],
        out_specs=pl.BlockSpec(memory_space=pltpu.VMEM),
        out_shape=jax.ShapeDtypeStruct((N, C), jnp.float32),
        scratch_shapes=[
            pltpu.VMEM((NBUF, BI, N), jnp.float32),
            pltpu.VMEM((N, H), jnp.float32),
            pltpu.VMEM((N, C), jnp.float32),
            pltpu.SemaphoreType.DMA((NBUF,)),
        ],
    )(adj, x, W1, b1r, W2, b2r)
    return out
